# baseline (speedup 1.0000x reference)
import jax
import jax.numpy as jnp
from jax import lax
from jax.experimental import pallas as pl
from jax.experimental.pallas import tpu as pltpu
from jax.experimental.pallas import tpu_sc as plsc

N = 50000
E = 800000
K = 4
HID = 32
OUT = 16
DIM_PE = 16
DIM_EMB = 128
DIM_IN = 128

NSUB = 16
NW = 32
N_PAD = 51200
CH_ROWS = 8
CHUNK_V = 160
CHUNK_E = CHUNK_V * 16
NCHUNKS = 384
E_PAD = (NCHUNKS + 3) * CHUNK_E


def _make_round(nch, nt):
    cpt = NCHUNKS // nt
    nquads = cpt // 4

    def body(tab_hbm, z_hbm, src_hbm, dst_hbm, out_hbm,
             table_v, acc_v, *bufsem):
        sbufs = bufsem[0:4]
        dbufs = bufsem[4:8]
        ssems = bufsem[8:12]
        dsems = bufsem[12:16]
        c = lax.axis_index("c")
        s = lax.axis_index("s")
        wid = c * NSUB + s
        ch = jnp.minimum(wid // nt, nch - 1)
        p = wid % nt
        base_c = p * cpt
        pltpu.sync_copy(tab_hbm.at[ch], table_v)
        pltpu.sync_copy(z_hbm, acc_v)

        def start(r, ci):
            e0 = (base_c + ci) * CHUNK_E
            pltpu.async_copy(src_hbm.at[pl.ds(e0, CHUNK_E)], sbufs[r],
                             ssems[r])
            pltpu.async_copy(dst_hbm.at[pl.ds(e0, CHUNK_E)], dbufs[r],
                             dsems[r])

        def wait(r):
            pltpu.make_async_copy(
                src_hbm.at[pl.ds(0, CHUNK_E)], sbufs[r], ssems[r]).wait()
            pltpu.make_async_copy(
                dst_hbm.at[pl.ds(0, CHUNK_E)], dbufs[r], dsems[r]).wait()

        def compute(r):
            buf_s, buf_d = sbufs[r], dbufs[r]
            for v0 in range(0, CHUNK_V, 8):
                sidx = [buf_s[pl.ds((v0 + i) * 16, 16)] for i in range(8)]
                didx = [buf_d[pl.ds((v0 + i) * 16, 16)] for i in range(8)]
                vals = [plsc.load_gather(table_v, [sx]) for sx in sidx]
                for i in range(8):
                    plsc.addupdate_scatter(acc_v, [didx[i]], vals[i])

        start(0, 0)
        start(1, 1)
        start(2, 2)

        def quad(j, carry):
            ci = 4 * j
            for r in range(4):
                start((r + 3) % 4, ci + r + 3)
                wait(r)
                compute(r)
            return carry

        lax.fori_loop(0, nquads, quad, 0)
        wait(0)
        wait(1)
        wait(2)
        pltpu.sync_copy(acc_v, out_hbm.at[wid])

    return pl.kernel(
        body,
        out_type=jax.ShapeDtypeStruct((NW, N_PAD), jnp.float32),
        mesh=plsc.VectorSubcoreMesh(core_axis_name="c", subcore_axis_name="s"),
        scratch_types=(
            pltpu.VMEM((N_PAD,), jnp.float32),
            pltpu.VMEM((N_PAD,), jnp.float32),
        ) + (pltpu.VMEM((CHUNK_E,), jnp.int32),) * 8
          + (pltpu.SemaphoreType.DMA,) * 8,
        compiler_params=pltpu.CompilerParams(needs_layout_passes=False),
    )


_BLK = 512
_GRID = N_PAD // _BLK


def _combine_body(p_ref, t_ref, o_ref):
    p = p_ref[...]
    aggs = [jnp.sum(p[ch * 6:(ch + 1) * 6, :], axis=0, keepdims=True)
            for ch in range(5)]
    aggs.append(jnp.zeros((3, p.shape[1]), jnp.float32))
    o_ref[...] = t_ref[...] + jnp.concatenate(aggs, axis=0)


def _combine(partials, tables):
    pspec = pl.BlockSpec((NW, _BLK), lambda i: (0, i))
    tspec = pl.BlockSpec((CH_ROWS, _BLK), lambda i: (0, i))
    return pl.pallas_call(
        _combine_body,
        grid=(_GRID,),
        in_specs=[pspec, tspec],
        out_specs=tspec,
        out_shape=jax.ShapeDtypeStruct((CH_ROWS, N_PAD), jnp.float32),
    )(partials, tables)


def _epilogue_body(x_ref, t2_ref, t3_ref, p_ref, sm_ref, g2t_ref, wxp_ref,
                   wrs_ref, bf_ref, o_ref):
    p = p_ref[...]
    t2 = t2_ref[...]
    t3 = t3_ref[...]
    sm = sm_ref[...]
    wc = sm[:, 0:1]
    u1 = sm[:, 1:2]
    u2 = sm[:, 2:3]
    b2a = sm[:, 3:4]
    br0e = sm[:, 4:5]
    d = t2[4:5, :]
    d2 = t3[4:5, :]
    cmat = u1 * d2 + u2 * d + b2a
    acc = jnp.zeros((HID, p.shape[1]), jnp.float32)
    for k in range(K):
        a3k = t3[k:k + 1, :] + jnp.sum(p[k * 8:(k + 1) * 8, :], axis=0,
                                       keepdims=True)
        t = wc * a3k
        sk = jax.nn.relu(cmat + t) + jax.nn.relu(cmat - t)
        acc = acc + jnp.dot(g2t_ref[k * HID:(k + 1) * HID, :], sk,
                            preferred_element_type=jnp.float32,
                            precision=lax.Precision.HIGHEST)
    q = jax.nn.relu(acc + br0e)
    pe = lax.dot_general(q, wrs_ref[...], (((0,), (0,)), ((), ())),
                         preferred_element_type=jnp.float32,
                         precision=lax.Precision.HIGHEST)
    o_ref[...] = (
        jnp.dot(x_ref[...], wxp_ref[...], preferred_element_type=jnp.float32,
                precision=lax.Precision.HIGHEST)
        + pe + bf_ref[...][0:1, :]
    )


def _epilogue(x_pad, t2, t3, partials, sm, g2t, wxp, wrs, bf):
    rowspec = pl.BlockSpec((_BLK, DIM_EMB), lambda i: (i, 0))
    tspec = pl.BlockSpec((CH_ROWS, _BLK), lambda i: (0, i))
    pspec = pl.BlockSpec((NW, _BLK), lambda i: (0, i))
    full = lambda r, w: pl.BlockSpec((r, w), lambda i: (0, 0))
    return pl.pallas_call(
        _epilogue_body,
        grid=(_GRID,),
        in_specs=[
            rowspec, tspec, tspec, pspec,
            full(HID, 8), full(K * HID, HID), full(DIM_IN, DIM_EMB),
            full(HID, DIM_EMB), full(8, DIM_EMB),
        ],
        out_specs=rowspec,
        out_shape=jax.ShapeDtypeStruct((N_PAD, DIM_EMB), jnp.float32),
    )(x_pad, t2, t3, partials, sm, g2t, wxp, wrs, bf)


def kernel(x, eigvecs, edge_index, batch_index, Wx, bx, W0, b0, W1, b1,
           W2a, b2a, W2b, b2b, Wr0, br0, Wr1, br1):
    f32 = jnp.float32
    wc = (W0 @ W1 @ W2a)[0]
    u1 = b0 @ W1 @ W2a
    u2 = b1 @ W2a
    wr0k = Wr0.reshape(K, OUT, HID)
    g2t = jnp.concatenate(
        [(W2b @ wr0k[k]).T for k in range(K)], axis=0)
    br0e = br0 + (2.0 * b2b) @ wr0k.sum(0)
    sm = jnp.stack([wc, u1, u2, b2a, br0e,
                    jnp.zeros_like(wc), jnp.zeros_like(wc),
                    jnp.zeros_like(wc)], axis=1)
    wxp = jnp.concatenate([Wx, jnp.zeros((DIM_IN, DIM_PE), f32)], 1)
    wrs = jnp.concatenate(
        [jnp.zeros((HID, DIM_EMB - DIM_PE), f32), Wr1], 1)
    bf = jnp.tile(jnp.concatenate([bx, br1])[None, :], (8, 1))

    z = jnp.where(jnp.isnan(eigvecs), 0.0, eigvecs)
    t1 = jnp.zeros((CH_ROWS, N_PAD), f32)
    t1 = t1.at[:K, :N].set(z.T)
    t1 = t1.at[K, :N].set(1.0)
    zeros_col = jnp.zeros((N_PAD,), f32)
    pad_idx = jnp.full((E_PAD - E,), N, jnp.int32)
    src = jnp.concatenate([edge_index[0], pad_idx])
    dst = jnp.concatenate([edge_index[1], pad_idx])
    x_pad = jnp.concatenate(
        [x, jnp.zeros((N_PAD - N, DIM_IN), f32)], axis=0)

    round5 = _make_round(5, 6)
    round4 = _make_round(4, 8)

    p1 = round5(t1, zeros_col, src, dst)
    t2 = _combine(p1, t1)
    p2 = round5(t2, zeros_col, src, dst)
    t3 = _combine(p2, t2)
    p3 = round4(t3, zeros_col, src, dst)

    out = _epilogue(x_pad, t2, t3, p3, sm, g2t, wxp, wrs, bf)
    return out[:N]


# speedup vs baseline: 1.4500x; 1.4500x over previous
import jax
import jax.numpy as jnp
from jax import lax
from jax.experimental import pallas as pl
from jax.experimental.pallas import tpu as pltpu
from jax.experimental.pallas import tpu_sc as plsc

N = 50000
E = 800000
K = 4
HID = 32
OUT = 16
DIM_PE = 16
DIM_EMB = 128
DIM_IN = 128

NSUB = 16
NW = 32
N_PAD = 51200
CH_ROWS = 8
CHUNK_V = 160
CHUNK_E = CHUNK_V * 16
NCHUNKS = 336
E_PAD = (NCHUNKS + 2) * CHUNK_E


def _make_round(nch, nt):
    cpt = NCHUNKS // nt
    npairs = cpt // 2

    def body(tab_hbm, z_hbm, src_hbm, dst_hbm, out_hbm,
             table_v, acc_v, s0, d0, s1, d1, ss0, sd0, ss1, sd1):
        c = lax.axis_index("c")
        s = lax.axis_index("s")
        wid = c * NSUB + s
        ch = jnp.minimum(wid // nt, nch - 1)
        p = wid % nt
        base_c = p * cpt
        pltpu.sync_copy(tab_hbm.at[ch], table_v)
        pltpu.sync_copy(z_hbm, acc_v)

        def start(buf_s, buf_d, sem_s, sem_d, ci):
            e0 = (base_c + ci) * CHUNK_E
            pltpu.async_copy(src_hbm.at[pl.ds(e0, CHUNK_E)], buf_s, sem_s)
            pltpu.async_copy(dst_hbm.at[pl.ds(e0, CHUNK_E)], buf_d, sem_d)

        def wait(buf_s, buf_d, sem_s, sem_d):
            pltpu.make_async_copy(
                src_hbm.at[pl.ds(0, CHUNK_E)], buf_s, sem_s).wait()
            pltpu.make_async_copy(
                dst_hbm.at[pl.ds(0, CHUNK_E)], buf_d, sem_d).wait()

        def compute(buf_s, buf_d):
            for v0 in range(0, CHUNK_V, 8):
                sidx = [buf_s[pl.ds((v0 + i) * 16, 16)] for i in range(8)]
                didx = [buf_d[pl.ds((v0 + i) * 16, 16)] for i in range(8)]
                vals = [plsc.load_gather(table_v, [sx]) for sx in sidx]
                for i in range(8):
                    plsc.addupdate_scatter(acc_v, [didx[i]], vals[i])

        start(s0, d0, ss0, sd0, 0)

        def pair(j, carry):
            start(s1, d1, ss1, sd1, 2 * j + 1)
            wait(s0, d0, ss0, sd0)
            compute(s0, d0)
            start(s0, d0, ss0, sd0, 2 * j + 2)
            wait(s1, d1, ss1, sd1)
            compute(s1, d1)
            return carry

        lax.fori_loop(0, npairs, pair, 0)
        wait(s0, d0, ss0, sd0)
        pltpu.sync_copy(acc_v, out_hbm.at[wid])

    return pl.kernel(
        body,
        out_type=jax.ShapeDtypeStruct((NW, N_PAD), jnp.float32),
        mesh=plsc.VectorSubcoreMesh(core_axis_name="c", subcore_axis_name="s"),
        scratch_types=(
            pltpu.VMEM((N_PAD,), jnp.float32),
            pltpu.VMEM((N_PAD,), jnp.float32),
            pltpu.VMEM((CHUNK_E,), jnp.int32),
            pltpu.VMEM((CHUNK_E,), jnp.int32),
            pltpu.VMEM((CHUNK_E,), jnp.int32),
            pltpu.VMEM((CHUNK_E,), jnp.int32),
            pltpu.SemaphoreType.DMA,
            pltpu.SemaphoreType.DMA,
            pltpu.SemaphoreType.DMA,
            pltpu.SemaphoreType.DMA,
        ),
        compiler_params=pltpu.CompilerParams(needs_layout_passes=False),
    )


_BLK = 512
_GRID = N_PAD // _BLK


def _combine_body(p_ref, t_ref, o_ref):
    p = p_ref[...]
    aggs = [jnp.sum(p[ch * 6:(ch + 1) * 6, :], axis=0, keepdims=True)
            for ch in range(5)]
    aggs.append(jnp.zeros((3, p.shape[1]), jnp.float32))
    o_ref[...] = t_ref[...] + jnp.concatenate(aggs, axis=0)


def _combine(partials, tables):
    pspec = pl.BlockSpec((NW, _BLK), lambda i: (0, i))
    tspec = pl.BlockSpec((CH_ROWS, _BLK), lambda i: (0, i))
    return pl.pallas_call(
        _combine_body,
        grid=(_GRID,),
        in_specs=[pspec, tspec],
        out_specs=tspec,
        out_shape=jax.ShapeDtypeStruct((CH_ROWS, N_PAD), jnp.float32),
    )(partials, tables)


def _epilogue_body(x_ref, t2_ref, t3_ref, p_ref, sm_ref, g2t_ref, wxp_ref,
                   wrs_ref, bf_ref, o_ref):
    p = p_ref[...]
    t2 = t2_ref[...]
    t3 = t3_ref[...]
    sm = sm_ref[...]
    wc = sm[:, 0:1]
    u1 = sm[:, 1:2]
    u2 = sm[:, 2:3]
    b2a = sm[:, 3:4]
    br0e = sm[:, 4:5]
    d = t2[4:5, :]
    d2 = t3[4:5, :]
    cmat = u1 * d2 + u2 * d + b2a
    acc = jnp.zeros((HID, p.shape[1]), jnp.float32)
    for k in range(K):
        a3k = t3[k:k + 1, :] + jnp.sum(p[k * 8:(k + 1) * 8, :], axis=0,
                                       keepdims=True)
        t = wc * a3k
        sk = jax.nn.relu(cmat + t) + jax.nn.relu(cmat - t)
        acc = acc + jnp.dot(g2t_ref[k * HID:(k + 1) * HID, :], sk,
                            preferred_element_type=jnp.float32,
                            precision=lax.Precision.HIGHEST)
    q = jax.nn.relu(acc + br0e)
    pe = lax.dot_general(q, wrs_ref[...], (((0,), (0,)), ((), ())),
                         preferred_element_type=jnp.float32,
                         precision=lax.Precision.HIGHEST)
    o_ref[...] = (
        jnp.dot(x_ref[...], wxp_ref[...], preferred_element_type=jnp.float32,
                precision=lax.Precision.HIGHEST)
        + pe + bf_ref[...][0:1, :]
    )


def _epilogue(x_pad, t2, t3, partials, sm, g2t, wxp, wrs, bf):
    rowspec = pl.BlockSpec((_BLK, DIM_EMB), lambda i: (i, 0))
    tspec = pl.BlockSpec((CH_ROWS, _BLK), lambda i: (0, i))
    pspec = pl.BlockSpec((NW, _BLK), lambda i: (0, i))
    full = lambda r, w: pl.BlockSpec((r, w), lambda i: (0, 0))
    return pl.pallas_call(
        _epilogue_body,
        grid=(_GRID,),
        in_specs=[
            rowspec, tspec, tspec, pspec,
            full(HID, 8), full(K * HID, HID), full(DIM_IN, DIM_EMB),
            full(HID, DIM_EMB), full(8, DIM_EMB),
        ],
        out_specs=rowspec,
        out_shape=jax.ShapeDtypeStruct((N_PAD, DIM_EMB), jnp.float32),
    )(x_pad, t2, t3, partials, sm, g2t, wxp, wrs, bf)


def kernel(x, eigvecs, edge_index, batch_index, Wx, bx, W0, b0, W1, b1,
           W2a, b2a, W2b, b2b, Wr0, br0, Wr1, br1):
    f32 = jnp.float32
    wc = (W0 @ W1 @ W2a)[0]
    u1 = b0 @ W1 @ W2a
    u2 = b1 @ W2a
    wr0k = Wr0.reshape(K, OUT, HID)
    g2t = jnp.concatenate(
        [(W2b @ wr0k[k]).T for k in range(K)], axis=0)
    br0e = br0 + (2.0 * b2b) @ wr0k.sum(0)
    sm = jnp.stack([wc, u1, u2, b2a, br0e,
                    jnp.zeros_like(wc), jnp.zeros_like(wc),
                    jnp.zeros_like(wc)], axis=1)
    wxp = jnp.concatenate([Wx, jnp.zeros((DIM_IN, DIM_PE), f32)], 1)
    wrs = jnp.concatenate(
        [jnp.zeros((HID, DIM_EMB - DIM_PE), f32), Wr1], 1)
    bf = jnp.tile(jnp.concatenate([bx, br1])[None, :], (8, 1))

    z = jnp.where(jnp.isnan(eigvecs), 0.0, eigvecs)
    t1 = jnp.zeros((CH_ROWS, N_PAD), f32)
    t1 = t1.at[:K, :N].set(z.T)
    t1 = t1.at[K, :N].set(1.0)
    zeros_col = jnp.zeros((N_PAD,), f32)
    pad_idx = jnp.full((E_PAD - E,), N, jnp.int32)
    src = jnp.concatenate([edge_index[0], pad_idx])
    dst = jnp.concatenate([edge_index[1], pad_idx])
    x_pad = jnp.concatenate(
        [x, jnp.zeros((N_PAD - N, DIM_IN), f32)], axis=0)

    round5 = _make_round(5, 6)
    round4 = _make_round(4, 8)

    p1 = round5(t1, zeros_col, src, dst)
    t2 = _combine(p1, t1)
    p2 = round5(t2, zeros_col, src, dst)
    t3 = _combine(p2, t2)
    p3 = round4(t3, zeros_col, src, dst)

    out = _epilogue(x_pad, t2, t3, p3, sm, g2t, wxp, wrs, bf)
    return out[:N]
